# Initial kernel scaffold; baseline (speedup 1.0000x reference)
import functools
import numpy as np

import jax
import jax.numpy as jnp
from jax.experimental import pallas as pl
from jax.experimental.pallas import tpu as pltpu

NT = 8; NQ = 128; NK = 10000; DF = 128; TD = 64; K = 16
RS = (0.5, 1.0); MAXT = 1.0; NENC = 10000.0; ANG = 1.0; LIN = 1.0
NKP = 10112
PADX = 1.0e18


def _dotT(a, b):
    return jax.lax.dot_general(a, b, (((1,), (1,)), ((), ())),
                               preferred_element_type=jnp.float32)


def _dot(a, b):
    return jax.lax.dot_general(a, b, (((1,), (0,)), ((), ())),
                               preferred_element_type=jnp.float32)


def _body(Ts_ref, te_ref, kxT0_ref, kf0_ref, kxT1_ref, kf1_ref, Pext_ref,
          qf_ref, qw_ref, Wq1_ref, bq1_ref, Wq2_ref, bq2_ref,
          W0_ref, U0_ref, W1_ref, U1_ref, aout_ref, wgt0, wgt1, cbuf):
    t = pl.program_id(0)

    u0 = Ts_ref[t, 0]; u1 = Ts_ref[t, 1]; u2 = Ts_ref[t, 2]; u3 = Ts_ref[t, 3]
    inv = jax.lax.rsqrt(u0 * u0 + u1 * u1 + u2 * u2 + u3 * u3)
    w = u0 * inv; x = u1 * inv; y = u2 * inv; z = u3 * inv
    R = ((1 - 2 * (y * y + z * z), 2 * (x * y - w * z), 2 * (x * z + w * y)),
         (2 * (x * y + w * z), 1 - 2 * (x * x + z * z), 2 * (y * z - w * x)),
         (2 * (x * z - w * y), 2 * (y * z + w * x), 1 - 2 * (x * x + y * y)))
    tr = (Ts_ref[t, 4], Ts_ref[t, 5], Ts_ref[t, 6])

    p0 = Pext_ref[:, 0:1]; p1 = Pext_ref[:, 1:2]; p2 = Pext_ref[:, 2:3]
    xc = [p0 * R[a][0] + p1 * R[a][1] + p2 * R[a][2] + tr[a] for a in range(3)]

    aggs = []
    for kxT_ref, kf_ref, wgt, r in ((kxT0_ref, kf0_ref, wgt0, RS[0]),
                                    (kxT1_ref, kf1_ref, wgt1, RS[1])):
        d2 = ((xc[0] - kxT_ref[0:1, :]) ** 2
              + (xc[1] - kxT_ref[1:2, :]) ** 2
              + (xc[2] - kxT_ref[2:3, :]) ** 2)
        prev = jnp.full((NQ, 1), -jnp.inf, dtype=jnp.float32)
        for _ in range(K):
            prev = jnp.min(jnp.where(d2 > prev, d2, jnp.inf), axis=1,
                           keepdims=True)
        wgt[...] = jnp.where(d2 <= prev,
                             jnp.exp(d2 * (-1.0 / (r * r))), 0.0)
        aggs.append(_dot(wgt[...], kf_ref[...]))

    h = _dot(te_ref[...], Wq1_ref[...]) + bq1_ref[...]
    h = h / (1.0 + jnp.exp(-h))
    qtemb = _dot(h, Wq2_ref[...]) + bq2_ref[...]
    ffU = _dot(qtemb, U0_ref[...] + U1_ref[...])

    out = _dot(aggs[0], W0_ref[...]) + _dot(aggs[1], W1_ref[...]) + ffU
    resid = out - qf_ref[...]

    gc = [jnp.zeros((NQ, 1), dtype=jnp.float32) for _ in range(3)]
    for kxT_ref, kf_ref, wgt, W_ref, r in ((kxT0_ref, kf0_ref, wgt0, W0_ref, RS[0]),
                                           (kxT1_ref, kf1_ref, wgt1, W1_ref, RS[1])):
        rW = _dotT(resid, W_ref[...])
        cbuf[...] = _dotT(rW, kf_ref[...])
        m = wgt[...] * cbuf[...]
        rowsum = jnp.sum(m, axis=1, keepdims=True)
        mk = _dotT(m, kxT_ref[...])
        s = -2.0 / (r * r)
        for a in range(3):
            gc[a] = gc[a] + s * (rowsum * xc[a] - mk[:, a:a + 1])

    qw = qw_ref[...]
    G = jnp.concatenate([gc[0], gc[1], gc[2],
                         jnp.zeros((NQ, 5), dtype=jnp.float32)], axis=1)
    G = G * (qw * (2.0 / DF))
    aout_ref[0] = jax.lax.dot_general(G, Pext_ref[...], (((0,), (0,)), ((), ())),
                                      preferred_element_type=jnp.float32)


def _qapply_local(q, p):
    w = q[..., 0:1]; v = q[..., 1:]
    t = 2.0 * jnp.cross(v, p)
    return p + w * t + jnp.cross(v, t)


@jax.jit
def kernel(Ts, time, key_x0, key_f0, key_x1, key_f1, query_x, query_f,
           query_w, Wq1, bq1, Wq2, bq2, W0, U0, W1, U1):
    f32 = jnp.float32
    half = TD // 2
    freqs = jnp.exp(jnp.arange(half, dtype=f32) * (-np.log(NENC) / (half - 1)))
    a = (time / MAXT)[:, None] * freqs[None, :]
    te = jnp.concatenate([jnp.sin(a), jnp.cos(a)], axis=-1)

    def padkx(kx):
        kxT = jnp.full((8, NKP), PADX, dtype=f32)
        return kxT.at[0:3, :NK].set(kx.T)
    kxT0 = padkx(key_x0); kxT1 = padkx(key_x1)
    kf0 = jnp.zeros((NKP, DF), f32).at[:NK].set(key_f0)
    kf1 = jnp.zeros((NKP, DF), f32).at[:NK].set(key_f1)
    Pext = jnp.zeros((NQ, 128), f32).at[:, 0:3].set(query_x).at[:, 3].set(1.0)

    full = lambda arr: pl.BlockSpec(arr.shape, lambda t: (0,) * arr.ndim)
    specs = [
        pl.BlockSpec(memory_space=pltpu.SMEM),
        pl.BlockSpec((1, TD), lambda t: (t, 0)),
    ]
    vmem_in = [kxT0, kf0, kxT1, kf1, Pext, query_f,
               query_w[:, None], Wq1, bq1[None], Wq2, bq2[None],
               W0, U0, W1, U1]
    specs += [full(v) for v in vmem_in]

    Aout = pl.pallas_call(
        _body,
        grid=(NT,),
        in_specs=specs,
        out_specs=pl.BlockSpec((1, 8, 128), lambda t: (t, 0, 0)),
        out_shape=jax.ShapeDtypeStruct((NT, 8, 128), f32),
        scratch_shapes=[pltpu.VMEM((NQ, NKP), f32)] * 3,
    )(Ts, te, *vmem_in)

    A = Aout[:, 0:3, 0:3]
    bvec = Aout[:, 0:3, 3]

    def S(T):
        qr = T[:, :4] / jnp.linalg.norm(T[:, :4], axis=-1, keepdims=True)
        e = jnp.eye(3, dtype=f32)
        Rcols = _qapply_local(qr[:, None, :], e[None, :, :])
        return jnp.sum(A * jnp.swapaxes(Rcols, 1, 2)) + jnp.sum(bvec * T[:, 4:])

    grad = -jax.grad(S)(Ts)

    qi = np.array([[1, 2, 3], [0, 3, 2], [3, 0, 1], [2, 1, 0]])
    qfac = jnp.array([[-0.5, -0.5, -0.5], [0.5, -0.5, 0.5],
                      [0.5, 0.5, -0.5], [-0.5, 0.5, 0.5]], dtype=f32)
    L = Ts[:, qi] * qfac
    ang_vel = jnp.einsum('tia,ti->ta', L, grad[:, :4]) * ANG
    qrn = Ts[:, :4] / jnp.linalg.norm(Ts[:, :4], axis=-1, keepdims=True)
    qinv = qrn * jnp.array([1.0, -1.0, -1.0, -1.0], dtype=f32)
    lin_vel = _qapply_local(qinv, grad[:, 4:]) * LIN
    return ang_vel, lin_vel


# speedup vs baseline: 1.9731x; 1.9731x over previous
import functools
import numpy as np

import jax
import jax.numpy as jnp
from jax.experimental import pallas as pl
from jax.experimental.pallas import tpu as pltpu

NT = 8; NQ = 128; NK = 10000; DF = 128; TD = 64; K = 16
RS = (0.5, 1.0); MAXT = 1.0; NENC = 10000.0; ANG = 1.0; LIN = 1.0
NKP = 10112
PADX = 1.0e18


_PREC = jax.lax.Precision.HIGHEST


def _dotT(a, b):
    return jax.lax.dot_general(a, b, (((1,), (1,)), ((), ())),
                               precision=_PREC,
                               preferred_element_type=jnp.float32)


def _dot(a, b):
    return jax.lax.dot_general(a, b, (((1,), (0,)), ((), ())),
                               precision=_PREC,
                               preferred_element_type=jnp.float32)


def _body(Ts_ref, te_ref, kxT0_ref, kf0_ref, kxT1_ref, kf1_ref, Pext_ref,
          qf_ref, qw_ref, Wq1_ref, bq1_ref, Wq2_ref, bq2_ref,
          W0_ref, U0_ref, W1_ref, U1_ref, aout_ref, wgt0, wgt1, cbuf):
    t = pl.program_id(0)

    u0 = Ts_ref[t, 0]; u1 = Ts_ref[t, 1]; u2 = Ts_ref[t, 2]; u3 = Ts_ref[t, 3]
    inv = jax.lax.rsqrt(u0 * u0 + u1 * u1 + u2 * u2 + u3 * u3)
    w = u0 * inv; x = u1 * inv; y = u2 * inv; z = u3 * inv
    R = ((1 - 2 * (y * y + z * z), 2 * (x * y - w * z), 2 * (x * z + w * y)),
         (2 * (x * y + w * z), 1 - 2 * (x * x + z * z), 2 * (y * z - w * x)),
         (2 * (x * z - w * y), 2 * (y * z + w * x), 1 - 2 * (x * x + y * y)))
    tr = (Ts_ref[t, 4], Ts_ref[t, 5], Ts_ref[t, 6])

    p0 = Pext_ref[:, 0:1]; p1 = Pext_ref[:, 1:2]; p2 = Pext_ref[:, 2:3]
    xc = [p0 * R[a][0] + p1 * R[a][1] + p2 * R[a][2] + tr[a] for a in range(3)]

    aggs = []
    for kxT_ref, kf_ref, wgt, r in ((kxT0_ref, kf0_ref, wgt0, RS[0]),
                                    (kxT1_ref, kf1_ref, wgt1, RS[1])):
        d2 = ((xc[0] - kxT_ref[0:1, :]) ** 2
              + (xc[1] - kxT_ref[1:2, :]) ** 2
              + (xc[2] - kxT_ref[2:3, :]) ** 2)
        prev = jnp.full((NQ, 1), -jnp.inf, dtype=jnp.float32)
        for _ in range(K):
            prev = jnp.min(jnp.where(d2 > prev, d2, jnp.inf), axis=1,
                           keepdims=True)
        wgt[...] = jnp.where(d2 <= prev,
                             jnp.exp(d2 * (-1.0 / (r * r))), 0.0)
        aggs.append(_dot(wgt[...], kf_ref[...]))

    h = _dot(te_ref[0], Wq1_ref[...]) + bq1_ref[...]
    h = h / (1.0 + jnp.exp(-h))
    qtemb = _dot(h, Wq2_ref[...]) + bq2_ref[...]
    ffU = _dot(qtemb, U0_ref[...] + U1_ref[...])

    out = _dot(aggs[0], W0_ref[...]) + _dot(aggs[1], W1_ref[...]) + ffU
    resid = out - qf_ref[...]

    gc = [jnp.zeros((NQ, 1), dtype=jnp.float32) for _ in range(3)]
    for kxT_ref, kf_ref, wgt, W_ref, r in ((kxT0_ref, kf0_ref, wgt0, W0_ref, RS[0]),
                                           (kxT1_ref, kf1_ref, wgt1, W1_ref, RS[1])):
        rW = _dotT(resid, W_ref[...])
        cbuf[...] = _dotT(rW, kf_ref[...])
        m = wgt[...] * cbuf[...]
        rowsum = jnp.sum(m, axis=1, keepdims=True)
        mk = _dotT(m, kxT_ref[...])
        s = -2.0 / (r * r)
        for a in range(3):
            gc[a] = gc[a] + s * (rowsum * xc[a] - mk[:, a:a + 1])

    qw = qw_ref[...]
    G = jnp.concatenate([gc[0], gc[1], gc[2],
                         jnp.zeros((NQ, 5), dtype=jnp.float32)], axis=1)
    G = G * (qw * (2.0 / DF))
    aout_ref[0] = jax.lax.dot_general(G, Pext_ref[...], (((0,), (0,)), ((), ())),
                                      precision=_PREC,
                                      preferred_element_type=jnp.float32)


def _qapply_local(q, p):
    w = q[..., 0:1]; v = q[..., 1:]
    t = 2.0 * jnp.cross(v, p)
    return p + w * t + jnp.cross(v, t)


@jax.jit
def kernel(Ts, time, key_x0, key_f0, key_x1, key_f1, query_x, query_f,
           query_w, Wq1, bq1, Wq2, bq2, W0, U0, W1, U1):
    f32 = jnp.float32
    half = TD // 2
    freqs = jnp.exp(jnp.arange(half, dtype=f32) * (-np.log(NENC) / (half - 1)))
    a = (time / MAXT)[:, None] * freqs[None, :]
    te = jnp.concatenate([jnp.sin(a), jnp.cos(a)], axis=-1)
    te = te[:, None, :]

    def padkx(kx):
        kxT = jnp.full((8, NKP), PADX, dtype=f32)
        return kxT.at[0:3, :NK].set(kx.T)
    kxT0 = padkx(key_x0); kxT1 = padkx(key_x1)
    kf0 = jnp.zeros((NKP, DF), f32).at[:NK].set(key_f0)
    kf1 = jnp.zeros((NKP, DF), f32).at[:NK].set(key_f1)
    Pext = jnp.zeros((NQ, 128), f32).at[:, 0:3].set(query_x).at[:, 3].set(1.0)

    full = lambda arr: pl.BlockSpec(arr.shape, lambda t: (0,) * arr.ndim)
    specs = [
        pl.BlockSpec(memory_space=pltpu.SMEM),
        pl.BlockSpec((1, 1, TD), lambda t: (t, 0, 0)),
    ]
    vmem_in = [kxT0, kf0, kxT1, kf1, Pext, query_f,
               query_w[:, None], Wq1, bq1[None], Wq2, bq2[None],
               W0, U0, W1, U1]
    specs += [full(v) for v in vmem_in]

    Aout = pl.pallas_call(
        _body,
        grid=(NT,),
        in_specs=specs,
        out_specs=pl.BlockSpec((1, 8, 128), lambda t: (t, 0, 0)),
        out_shape=jax.ShapeDtypeStruct((NT, 8, 128), f32),
        scratch_shapes=[pltpu.VMEM((NQ, NKP), f32)] * 3,
    )(Ts, te, *vmem_in)

    A = Aout[:, 0:3, 0:3]
    bvec = Aout[:, 0:3, 3]

    def S(T):
        qr = T[:, :4] / jnp.linalg.norm(T[:, :4], axis=-1, keepdims=True)
        e = jnp.eye(3, dtype=f32)
        Rcols = _qapply_local(qr[:, None, :], e[None, :, :])
        return jnp.sum(A * jnp.swapaxes(Rcols, 1, 2)) + jnp.sum(bvec * T[:, 4:])

    grad = -jax.grad(S)(Ts)

    qi = np.array([[1, 2, 3], [0, 3, 2], [3, 0, 1], [2, 1, 0]])
    qfac = jnp.array([[-0.5, -0.5, -0.5], [0.5, -0.5, 0.5],
                      [0.5, 0.5, -0.5], [-0.5, 0.5, 0.5]], dtype=f32)
    L = Ts[:, qi] * qfac
    ang_vel = jnp.einsum('tia,ti->ta', L, grad[:, :4]) * ANG
    qrn = Ts[:, :4] / jnp.linalg.norm(Ts[:, :4], axis=-1, keepdims=True)
    qinv = qrn * jnp.array([1.0, -1.0, -1.0, -1.0], dtype=f32)
    lin_vel = _qapply_local(qinv, grad[:, 4:]) * LIN
    return ang_vel, lin_vel
